# baseline (speedup 1.0000x reference)
import functools

import jax
import jax.numpy as jnp
from jax import lax
from jax.experimental import pallas as pl
from jax.experimental.pallas import tpu as pltpu
from jax.experimental.pallas import tpu_sc as plsc

DT = 0.1
OUT_SZ = 1000
PAR_SZ = 2

NC = 2
NS = 16
LANES = 16
C = 2048
RPC = C // 128
VPR = 128 // LANES


def _sc_spmm(rows2d, cols2d, w2d, y_state, in_flat, H, B, I, chunks_pw):
  total_chunks = chunks_pw
  DUMMY = B * H
  ACCB = B * H + 4096
  UCAP = C + LANES

  mesh = plsc.VectorSubcoreMesh(core_axis_name="c", subcore_axis_name="s")

  @functools.partial(
      pl.kernel,
      mesh=mesh,
      compiler_params=pltpu.CompilerParams(needs_layout_passes=False),
      out_type=[
          jax.ShapeDtypeStruct((NC, H), jnp.float32),
          jax.ShapeDtypeStruct((NC, B * H), jnp.float32),
      ],
      scratch_types=[
          pltpu.VMEM((H,), jnp.float32),
          pltpu.VMEM((I * B,), jnp.float32),
          pltpu.VMEM((2, RPC, 128), jnp.int32),
          pltpu.VMEM((2, RPC, 128), jnp.int32),
          pltpu.VMEM((2, RPC, 128), jnp.float32),
          pltpu.VMEM((RPC, 128), jnp.float32),
          pltpu.VMEM((UCAP,), jnp.int32),
          pltpu.VMEM((UCAP,), jnp.int32),
          pltpu.VMEM((UCAP,), jnp.float32),
          pltpu.VMEM((128,), jnp.int32),
          pltpu.VMEM((128,), jnp.float32),
          pltpu.VMEM((2048,), jnp.float32),
          pltpu.VMEM_SHARED((H,), jnp.float32),
          pltpu.VMEM_SHARED((ACCB,), jnp.float32),
          pltpu.SemaphoreType.DMA,
          pltpu.SemaphoreType.DMA,
          pltpu.SemaphoreType.DMA,
      ],
  )
  def k(rows_hbm, cols_hbm, w_hbm, y_hbm, in_hbm, accy_out, accbu_out,
        y_tab, in_tab, rows_v, cols_v, w_v, py, u_cols, u_rows, u_w,
        pu_idx, pu_val, zbuf, acc_y, acc_bu, in_sem0, in_sem1, sc_sem):
    core = lax.axis_index("c")
    sub = lax.axis_index("s")
    wid = core * NS + sub
    wrow = wid * total_chunks * RPC
    in_sems = (in_sem0, in_sem1)

    zero16f = jnp.zeros((LANES,), jnp.float32)
    zero16i = jnp.zeros((LANES,), jnp.int32)

    def _z(i, _):
      zbuf[pl.ds(i * LANES, LANES)] = zero16f
      return _
    lax.fori_loop(0, 2048 // LANES, _z, 0)

    def _zu(i, _):
      u_cols[pl.ds(i * LANES, LANES)] = zero16i
      u_rows[pl.ds(i * LANES, LANES)] = zero16i
      u_w[pl.ds(i * LANES, LANES)] = zero16f
      return _
    lax.fori_loop(0, UCAP // LANES, _zu, 0)

    ZB = 2048
    ystride = H // NS
    for t in range(ystride // ZB):
      pltpu.sync_copy(zbuf, acc_y.at[pl.ds(sub * ystride + t * ZB, ZB)])
    bstride = ACCB // NS
    for t in range(bstride // ZB):
      pltpu.sync_copy(zbuf, acc_bu.at[pl.ds(sub * bstride + t * ZB, ZB)])
    rem = bstride % ZB
    if rem:
      pltpu.sync_copy(zbuf.at[pl.ds(0, rem)],
                      acc_bu.at[pl.ds(sub * bstride + (bstride // ZB) * ZB,
                                      rem)])

    pltpu.sync_copy(y_hbm, y_tab)
    pltpu.sync_copy(in_hbm, in_tab)
    plsc.subcore_barrier()

    iota = lax.iota(jnp.int32, LANES)
    bl = jnp.bitwise_and(iota, B - 1)

    def issue_chunk(ck, par):
      base = wrow + ck * RPC
      pltpu.async_copy(rows_hbm.at[pl.ds(base, RPC)], rows_v.at[par],
                       in_sems[par])
      pltpu.async_copy(cols_hbm.at[pl.ds(base, RPC)], cols_v.at[par],
                       in_sems[par])
      pltpu.async_copy(w_hbm.at[pl.ds(base, RPC)], w_v.at[par], in_sems[par])

    def wait_chunk(ck, par):
      base = wrow + ck * RPC
      pltpu.make_async_copy(rows_hbm.at[pl.ds(base, RPC)], rows_v.at[par],
                            in_sems[par]).wait()
      pltpu.make_async_copy(cols_hbm.at[pl.ds(base, RPC)], cols_v.at[par],
                            in_sems[par]).wait()
      pltpu.make_async_copy(w_hbm.at[pl.ds(base, RPC)], w_v.at[par],
                            in_sems[par]).wait()

    issue_chunk(0, 0)

    def do_chunk(ck, par):
      wait_chunk(ck, par)

      @pl.when(ck + 1 < total_chunks)
      def _():
        issue_chunk(ck + 1, 1 - par)

      @plsc.parallel_loop(0, RPC * VPR, unroll=8)
      def _(i):
        j = jnp.right_shift(i, 3)
        sl = pl.ds(jnp.bitwise_and(i, VPR - 1) * LANES, LANES)
        cvec = cols_v[par, j, sl]
        wvec = w_v[par, j, sl]
        iy = cvec - I
        my = iy >= 0
        gy = plsc.load_gather(y_tab, [jnp.maximum(iy, 0)])
        py[j, sl] = wvec * jnp.where(my, gy, 0.0)

      for j in range(RPC):
        pltpu.async_copy(py.at[j], acc_y.at[rows_v.at[par, j]], sc_sem,
                         add=True)

      def rowscan(j, uc):
        macc = jnp.zeros((LANES,), jnp.bool_)
        for v in range(VPR):
          macc = jnp.logical_or(
              macc, cols_v[par, j, pl.ds(v * LANES, LANES)] < I)
        anyu = jnp.any(macc)

        def hit_row(uc2):
          for v in range(VPR):
            sl = pl.ds(v * LANES, LANES)
            cvec = cols_v[par, j, sl]
            mu = cvec < I

            def dov(uc3, cvec=cvec, mu=mu, sl=sl):
              plsc.store_compressed(u_cols.at[pl.ds(uc3, LANES)], cvec,
                                    mask=mu)
              plsc.store_compressed(u_rows.at[pl.ds(uc3, LANES)],
                                    rows_v[par, j, sl], mask=mu)
              plsc.store_compressed(u_w.at[pl.ds(uc3, LANES)],
                                    w_v[par, j, sl], mask=mu)
              return uc3 + jnp.sum(mu.astype(jnp.int32))
            uc2 = lax.cond(jnp.any(mu), dov, lambda u: u, uc2)
          return uc2
        return lax.cond(anyu, hit_row, lambda u: u, uc)
      ucnt = lax.fori_loop(0, RPC, rowscan, jnp.int32(0))

      def ublock(ub, _):
        for v in range(VPR):
          gl = v * LANES + iota
          eidx = ub * LANES + jnp.right_shift(gl, 3)
          valid = eidx < ucnt
          eidxc = jnp.where(valid, eidx, 0)
          c2 = plsc.load_gather(u_cols, [eidxc])
          r2 = plsc.load_gather(u_rows, [eidxc])
          w2 = plsc.load_gather(u_w, [eidxc])
          g = plsc.load_gather(in_tab, [c2 * B + bl])
          pu_val[pl.ds(v * LANES, LANES)] = jnp.where(valid, w2 * g, 0.0)
          pu_idx[pl.ds(v * LANES, LANES)] = jnp.where(
              valid, bl * H + r2, DUMMY + gl)
        pltpu.sync_copy(pu_val, acc_bu.at[pu_idx], add=True)
        return _
      nub = (ucnt + (LANES - 1)) // LANES
      lax.fori_loop(0, nub, ublock, 0)

      for j in range(RPC):
        pltpu.make_async_copy(py.at[j], acc_y.at[rows_v.at[par, j]],
                              sc_sem).wait()

    def loop2(i, _):
      do_chunk(2 * i, 0)
      do_chunk(2 * i + 1, 1)
      return _
    lax.fori_loop(0, total_chunks // 2, loop2, 0)

    plsc.subcore_barrier()

    pltpu.sync_copy(acc_y.at[pl.ds(sub * ystride, ystride)],
                    accy_out.at[core, pl.ds(sub * ystride, ystride)])
    bout = (B * H) // NS
    pltpu.sync_copy(acc_bu.at[pl.ds(sub * bout, bout)],
                    accbu_out.at[core, pl.ds(sub * bout, bout)])

  return k(rows2d, cols2d, w2d, y_state, in_flat)


def _tc_finale(accy, accbu3, x2, tau2, bias2, act2, H, B):
  BH = 8192
  grid = (H // BH,)

  def body(accy_ref, accbu_ref, x_ref, tau_ref, bias_ref, act_ref,
           nx_ref, ny_ref, e_ref):
    accv = accy_ref[0:1, :] + accy_ref[1:2, :]
    act_in = accbu_ref[0] + accbu_ref[1] + accv
    x = x_ref[...]
    dx = (act_in + bias_ref[...] - x) / tau_ref[...] * DT
    nx = x + dx
    nx_ref[...] = nx
    at = act_ref[...]
    ny = jnp.where(at == 0, jnp.tanh(nx),
                   jnp.where(at == 1, jax.nn.sigmoid(nx),
                             jnp.maximum(nx, 0.0)))
    ny_ref[...] = ny

    @pl.when(pl.program_id(0) == 0)
    def _():
      e_ref[...] = jnp.zeros((1, 1), jnp.float32)
    e_ref[...] += jnp.reshape(jnp.sum(jnp.abs(ny)), (1, 1))

  return pl.pallas_call(
      body,
      grid=grid,
      in_specs=[
          pl.BlockSpec((NC, BH), lambda i: (0, i)),
          pl.BlockSpec((NC, B, BH), lambda i: (0, 0, i)),
          pl.BlockSpec((1, BH), lambda i: (0, i)),
          pl.BlockSpec((1, BH), lambda i: (0, i)),
          pl.BlockSpec((1, BH), lambda i: (0, i)),
          pl.BlockSpec((1, BH), lambda i: (0, i)),
      ],
      out_specs=[
          pl.BlockSpec((B, BH), lambda i: (0, i)),
          pl.BlockSpec((B, BH), lambda i: (0, i)),
          pl.BlockSpec((1, 1), lambda i: (0, 0)),
      ],
      out_shape=[
          jax.ShapeDtypeStruct((B, H), jnp.float32),
          jax.ShapeDtypeStruct((B, H), jnp.float32),
          jax.ShapeDtypeStruct((1, 1), jnp.float32),
      ],
  )(accy, accbu3, x2, tau2, bias2, act2)


def kernel(inputs, tau, bias, weight_values, x_state, y_state, rows, cols,
           act_types):
  B, I = inputs.shape
  H = tau.shape[0]
  nnz = rows.shape[0]
  NW = NC * NS

  chunks_pw = -(-nnz // (NW * C))
  chunks_pw += chunks_pw % 2
  Np = NW * chunks_pw * C
  pad = Np - nnz
  rows32 = rows.astype(jnp.int32)
  cols32 = cols.astype(jnp.int32)
  w32 = weight_values.astype(jnp.float32)
  pad_rows = (jnp.arange(pad, dtype=jnp.int32) * 1031) % H
  rows_p = jnp.concatenate([rows32, pad_rows]).reshape(Np // 128, 128)
  cols_p = jnp.concatenate(
      [cols32, jnp.full((pad,), I, jnp.int32)]).reshape(Np // 128, 128)
  w_p = jnp.concatenate(
      [w32, jnp.zeros((pad,), jnp.float32)]).reshape(Np // 128, 128)
  in_flat = inputs.T.reshape(-1)

  accy, accbu = _sc_spmm(rows_p, cols_p, w_p, y_state, in_flat,
                         H, B, I, chunks_pw)
  accbu3 = accbu.reshape(NC, B, H)

  one = lambda v: v.reshape(1, H)
  new_x, new_y, e = _tc_finale(accy, accbu3, one(x_state), one(tau),
                               one(bias), one(act_types.astype(jnp.int32)),
                               H, B)
  lo = H - OUT_SZ - PAR_SZ
  outputs = new_y[:, lo:lo + OUT_SZ]
  energy = e[0, 0] * 0.001
  return (outputs, new_x, energy)


# speedup vs baseline: 1.9516x; 1.9516x over previous
import functools

import jax
import jax.numpy as jnp
from jax import lax
from jax.experimental import pallas as pl
from jax.experimental.pallas import tpu as pltpu
from jax.experimental.pallas import tpu_sc as plsc

DT = 0.1
OUT_SZ = 1000
PAR_SZ = 2

NC = 2
NS = 16
LANES = 16
C = 2048
RPC = C // 128
VPR = 128 // LANES


def _sc_spmm(rows2d, cols2d, w2d, y_state, in_flat, H, B, I, chunks_pw):
  total_chunks = chunks_pw
  DUMMY = B * H
  ACCB = B * H + 4096
  UCAP = C + LANES

  mesh = plsc.VectorSubcoreMesh(core_axis_name="c", subcore_axis_name="s")

  @functools.partial(
      pl.kernel,
      mesh=mesh,
      compiler_params=pltpu.CompilerParams(needs_layout_passes=False),
      out_type=[
          jax.ShapeDtypeStruct((NC, H), jnp.float32),
          jax.ShapeDtypeStruct((NC, B * H), jnp.float32),
      ],
      scratch_types=[
          pltpu.VMEM((H,), jnp.float32),
          pltpu.VMEM((I * B,), jnp.float32),
          pltpu.VMEM((2, RPC, 128), jnp.int32),
          pltpu.VMEM((2, RPC, 128), jnp.int32),
          pltpu.VMEM((2, RPC, 128), jnp.float32),
          pltpu.VMEM((RPC, 128), jnp.float32),
          pltpu.VMEM((UCAP,), jnp.int32),
          pltpu.VMEM((UCAP,), jnp.int32),
          pltpu.VMEM((UCAP,), jnp.float32),
          pltpu.VMEM((128,), jnp.int32),
          pltpu.VMEM((128,), jnp.float32),
          pltpu.VMEM((2048,), jnp.float32),
          pltpu.VMEM_SHARED((H,), jnp.float32),
          pltpu.VMEM_SHARED((ACCB,), jnp.float32),
          pltpu.SemaphoreType.DMA,
          pltpu.SemaphoreType.DMA,
          pltpu.SemaphoreType.DMA,
      ],
  )
  def k(rows_hbm, cols_hbm, w_hbm, y_hbm, in_hbm, accy_out, accbu_out,
        y_tab, in_tab, rows_v, cols_v, w_v, py, u_cols, u_rows, u_w,
        pu_idx, pu_val, zbuf, acc_y, acc_bu, in_sem0, in_sem1, sc_sem):
    core = lax.axis_index("c")
    sub = lax.axis_index("s")
    wid = core * NS + sub
    wrow = wid * total_chunks * RPC
    in_sems = (in_sem0, in_sem1)

    zero16f = jnp.zeros((LANES,), jnp.float32)
    zero16i = jnp.zeros((LANES,), jnp.int32)

    def _z(i, _):
      zbuf[pl.ds(i * LANES, LANES)] = zero16f
      return _
    lax.fori_loop(0, 2048 // LANES, _z, 0)

    def _zu(i, _):
      u_cols[pl.ds(i * LANES, LANES)] = zero16i
      u_rows[pl.ds(i * LANES, LANES)] = zero16i
      u_w[pl.ds(i * LANES, LANES)] = zero16f
      return _
    lax.fori_loop(0, UCAP // LANES, _zu, 0)

    ZB = 2048
    ystride = H // NS
    for t in range(ystride // ZB):
      pltpu.sync_copy(zbuf, acc_y.at[pl.ds(sub * ystride + t * ZB, ZB)])
    bstride = ACCB // NS
    for t in range(bstride // ZB):
      pltpu.sync_copy(zbuf, acc_bu.at[pl.ds(sub * bstride + t * ZB, ZB)])
    rem = bstride % ZB
    if rem:
      pltpu.sync_copy(zbuf.at[pl.ds(0, rem)],
                      acc_bu.at[pl.ds(sub * bstride + (bstride // ZB) * ZB,
                                      rem)])

    pltpu.sync_copy(y_hbm, y_tab)
    pltpu.sync_copy(in_hbm, in_tab)
    plsc.subcore_barrier()

    iota = lax.iota(jnp.int32, LANES)
    bl = jnp.bitwise_and(iota, B - 1)

    def issue_chunk(ck, par):
      base = wrow + ck * RPC
      pltpu.async_copy(rows_hbm.at[pl.ds(base, RPC)], rows_v.at[par],
                       in_sems[par])
      pltpu.async_copy(cols_hbm.at[pl.ds(base, RPC)], cols_v.at[par],
                       in_sems[par])
      pltpu.async_copy(w_hbm.at[pl.ds(base, RPC)], w_v.at[par], in_sems[par])

    def wait_chunk(ck, par):
      base = wrow + ck * RPC
      pltpu.make_async_copy(rows_hbm.at[pl.ds(base, RPC)], rows_v.at[par],
                            in_sems[par]).wait()
      pltpu.make_async_copy(cols_hbm.at[pl.ds(base, RPC)], cols_v.at[par],
                            in_sems[par]).wait()
      pltpu.make_async_copy(w_hbm.at[pl.ds(base, RPC)], w_v.at[par],
                            in_sems[par]).wait()

    issue_chunk(0, 0)

    def do_chunk(ck, par):
      wait_chunk(ck, par)

      @pl.when(ck + 1 < total_chunks)
      def _():
        issue_chunk(ck + 1, 1 - par)

      @plsc.parallel_loop(0, RPC * VPR, unroll=8,
                          carry=jnp.zeros((LANES,), jnp.int32))
      def ucntv(i, ucv):
        j = jnp.right_shift(i, 3)
        sl = pl.ds(jnp.bitwise_and(i, VPR - 1) * LANES, LANES)
        cvec = cols_v[par, j, sl]
        wvec = w_v[par, j, sl]
        iy = cvec - I
        my = iy >= 0
        gy = plsc.load_gather(y_tab, [jnp.maximum(iy, 0)])
        py[j, sl] = wvec * jnp.where(my, gy, 0.0)
        mu = jnp.logical_not(my)
        mui = mu.astype(jnp.int32)
        pos = ucv + plsc.cumsum(mui) - mui
        plsc.store_scatter(u_cols, [pos], cvec, mask=mu)
        plsc.store_scatter(u_rows, [pos], rows_v[par, j, sl], mask=mu)
        plsc.store_scatter(u_w, [pos], wvec, mask=mu)
        return ucv + plsc.all_reduce_population_count(mu)
      ucnt = jnp.max(ucntv)

      for j in range(RPC):
        pltpu.async_copy(py.at[j], acc_y.at[rows_v.at[par, j]], sc_sem,
                         add=True)

      def ublock(ub, _):
        for v in range(VPR):
          gl = v * LANES + iota
          eidx = ub * LANES + jnp.right_shift(gl, 3)
          valid = eidx < ucnt
          eidxc = jnp.where(valid, eidx, 0)
          c2 = plsc.load_gather(u_cols, [eidxc])
          r2 = plsc.load_gather(u_rows, [eidxc])
          w2 = plsc.load_gather(u_w, [eidxc])
          g = plsc.load_gather(in_tab, [c2 * B + bl])
          pu_val[pl.ds(v * LANES, LANES)] = jnp.where(valid, w2 * g, 0.0)
          pu_idx[pl.ds(v * LANES, LANES)] = jnp.where(
              valid, bl * H + r2, DUMMY + gl)
        pltpu.sync_copy(pu_val, acc_bu.at[pu_idx], add=True)
        return _
      nub = (ucnt + (LANES - 1)) // LANES
      lax.fori_loop(0, nub, ublock, 0)

      for j in range(RPC):
        pltpu.make_async_copy(py.at[j], acc_y.at[rows_v.at[par, j]],
                              sc_sem).wait()

    def loop2(i, _):
      do_chunk(2 * i, 0)
      do_chunk(2 * i + 1, 1)
      return _
    lax.fori_loop(0, total_chunks // 2, loop2, 0)

    plsc.subcore_barrier()

    pltpu.sync_copy(acc_y.at[pl.ds(sub * ystride, ystride)],
                    accy_out.at[core, pl.ds(sub * ystride, ystride)])
    bout = (B * H) // NS
    pltpu.sync_copy(acc_bu.at[pl.ds(sub * bout, bout)],
                    accbu_out.at[core, pl.ds(sub * bout, bout)])

  return k(rows2d, cols2d, w2d, y_state, in_flat)


def _tc_finale(accy, accbu3, x2, tau2, bias2, act2, H, B):
  BH = 8192
  grid = (H // BH,)

  def body(accy_ref, accbu_ref, x_ref, tau_ref, bias_ref, act_ref,
           nx_ref, ny_ref, e_ref):
    accv = accy_ref[0:1, :] + accy_ref[1:2, :]
    act_in = accbu_ref[0] + accbu_ref[1] + accv
    x = x_ref[...]
    dx = (act_in + bias_ref[...] - x) / tau_ref[...] * DT
    nx = x + dx
    nx_ref[...] = nx
    at = act_ref[...]
    ny = jnp.where(at == 0, jnp.tanh(nx),
                   jnp.where(at == 1, jax.nn.sigmoid(nx),
                             jnp.maximum(nx, 0.0)))
    ny_ref[...] = ny

    @pl.when(pl.program_id(0) == 0)
    def _():
      e_ref[...] = jnp.zeros((1, 1), jnp.float32)
    e_ref[...] += jnp.reshape(jnp.sum(jnp.abs(ny)), (1, 1))

  return pl.pallas_call(
      body,
      grid=grid,
      in_specs=[
          pl.BlockSpec((NC, BH), lambda i: (0, i)),
          pl.BlockSpec((NC, B, BH), lambda i: (0, 0, i)),
          pl.BlockSpec((1, BH), lambda i: (0, i)),
          pl.BlockSpec((1, BH), lambda i: (0, i)),
          pl.BlockSpec((1, BH), lambda i: (0, i)),
          pl.BlockSpec((1, BH), lambda i: (0, i)),
      ],
      out_specs=[
          pl.BlockSpec((B, BH), lambda i: (0, i)),
          pl.BlockSpec((B, BH), lambda i: (0, i)),
          pl.BlockSpec((1, 1), lambda i: (0, 0)),
      ],
      out_shape=[
          jax.ShapeDtypeStruct((B, H), jnp.float32),
          jax.ShapeDtypeStruct((B, H), jnp.float32),
          jax.ShapeDtypeStruct((1, 1), jnp.float32),
      ],
  )(accy, accbu3, x2, tau2, bias2, act2)


def kernel(inputs, tau, bias, weight_values, x_state, y_state, rows, cols,
           act_types):
  B, I = inputs.shape
  H = tau.shape[0]
  nnz = rows.shape[0]
  NW = NC * NS

  chunks_pw = -(-nnz // (NW * C))
  chunks_pw += chunks_pw % 2
  Np = NW * chunks_pw * C
  pad = Np - nnz
  rows32 = rows.astype(jnp.int32)
  cols32 = cols.astype(jnp.int32)
  w32 = weight_values.astype(jnp.float32)
  pad_rows = (jnp.arange(pad, dtype=jnp.int32) * 1031) % H
  rows_p = jnp.concatenate([rows32, pad_rows]).reshape(Np // 128, 128)
  cols_p = jnp.concatenate(
      [cols32, jnp.full((pad,), I, jnp.int32)]).reshape(Np // 128, 128)
  w_p = jnp.concatenate(
      [w32, jnp.zeros((pad,), jnp.float32)]).reshape(Np // 128, 128)
  in_flat = inputs.T.reshape(-1)

  accy, accbu = _sc_spmm(rows_p, cols_p, w_p, y_state, in_flat,
                         H, B, I, chunks_pw)
  accbu3 = accbu.reshape(NC, B, H)

  one = lambda v: v.reshape(1, H)
  new_x, new_y, e = _tc_finale(accy, accbu3, one(x_state), one(tau),
                               one(bias), one(act_types.astype(jnp.int32)),
                               H, B)
  lo = H - OUT_SZ - PAR_SZ
  outputs = new_y[:, lo:lo + OUT_SZ]
  energy = e[0, 0] * 0.001
  return (outputs, new_x, energy)


# speedup vs baseline: 2.1202x; 1.0864x over previous
import functools

import jax
import jax.numpy as jnp
from jax import lax
from jax.experimental import pallas as pl
from jax.experimental.pallas import tpu as pltpu
from jax.experimental.pallas import tpu_sc as plsc

DT = 0.1
OUT_SZ = 1000
PAR_SZ = 2

NC = 2
NS = 16
LANES = 16
C = 2048
RPC = C // 128
VPR = 128 // LANES


def _sc_spmm(rows2d, cols2d, w2d, rows_t, cols_t, w_t, tch,
             y_state, in_flat, H, B, I, chunks_pw):
  total_chunks = chunks_pw
  DUMMY = B * H
  ACCB = B * H + 4096
  UCAP = C + LANES

  mesh = plsc.VectorSubcoreMesh(core_axis_name="c", subcore_axis_name="s")

  @functools.partial(
      pl.kernel,
      mesh=mesh,
      compiler_params=pltpu.CompilerParams(needs_layout_passes=False),
      out_type=[
          jax.ShapeDtypeStruct((NC, H), jnp.float32),
          jax.ShapeDtypeStruct((NC * B, H), jnp.float32),
      ],
      scratch_types=[
          pltpu.VMEM((H + LANES,), jnp.float32),
          pltpu.VMEM((I * B,), jnp.float32),
          pltpu.VMEM((2, RPC, 128), jnp.int32),
          pltpu.VMEM((2, RPC, 128), jnp.int32),
          pltpu.VMEM((2, RPC, 128), jnp.float32),
          pltpu.VMEM((RPC, 128), jnp.float32),
          pltpu.VMEM((UCAP,), jnp.int32),
          pltpu.VMEM((16, 128), jnp.int32),
          pltpu.VMEM((16, 128), jnp.float32),
          pltpu.VMEM((2048,), jnp.float32),
          pltpu.VMEM_SHARED((H,), jnp.float32),
          pltpu.VMEM_SHARED((ACCB,), jnp.float32),
          pltpu.SemaphoreType.DMA,
          pltpu.SemaphoreType.DMA,
          pltpu.SemaphoreType.DMA,
      ],
  )
  def k(rows_hbm, cols_hbm, w_hbm, rows_th, cols_th, w_th,
        y_hbm, in_hbm, accy_out, accbu_out,
        y_tab, in_tab, rows_v, cols_v, w_v, py, u_pos,
        pu_idx, pu_val, zbuf, acc_y, acc_bu, in_sem0, in_sem1, sc_sem):
    core = lax.axis_index("c")
    sub = lax.axis_index("s")
    wid = core * NS + sub
    wrow = wid * total_chunks * RPC
    in_sems = (in_sem0, in_sem1)

    zero16f = jnp.zeros((LANES,), jnp.float32)
    zero16i = jnp.zeros((LANES,), jnp.int32)

    def _z(i, _):
      zbuf[pl.ds(i * LANES, LANES)] = zero16f
      return _
    lax.fori_loop(0, 2048 // LANES, _z, 0)

    def _zu(i, _):
      u_pos[pl.ds(i * LANES, LANES)] = zero16i
      return _
    lax.fori_loop(0, UCAP // LANES, _zu, 0)

    ZB = 2048
    ystride = H // NS
    for t in range(ystride // ZB):
      pltpu.sync_copy(zbuf, acc_y.at[pl.ds(sub * ystride + t * ZB, ZB)])
    bstride = ACCB // NS
    for t in range(bstride // ZB):
      pltpu.sync_copy(zbuf, acc_bu.at[pl.ds(sub * bstride + t * ZB, ZB)])
    rem = bstride % ZB
    if rem:
      pltpu.sync_copy(zbuf.at[pl.ds(0, rem)],
                      acc_bu.at[pl.ds(sub * bstride + (bstride // ZB) * ZB,
                                      rem)])

    pltpu.sync_copy(y_hbm, y_tab.at[pl.ds(0, H)])
    y_tab[pl.ds(H, LANES)] = zero16f
    pltpu.sync_copy(in_hbm, in_tab)
    plsc.subcore_barrier()

    iota = lax.iota(jnp.int32, LANES)
    bl = jnp.bitwise_and(iota, B - 1)

    def issue_chunk(ck, par):
      base = wrow + ck * RPC
      pltpu.async_copy(rows_hbm.at[pl.ds(base, RPC)], rows_v.at[par],
                       in_sems[par])
      pltpu.async_copy(cols_hbm.at[pl.ds(base, RPC)], cols_v.at[par],
                       in_sems[par])
      pltpu.async_copy(w_hbm.at[pl.ds(base, RPC)], w_v.at[par], in_sems[par])

    def wait_chunk(ck, par):
      base = wrow + ck * RPC
      pltpu.make_async_copy(rows_hbm.at[pl.ds(base, RPC)], rows_v.at[par],
                            in_sems[par]).wait()
      pltpu.make_async_copy(cols_hbm.at[pl.ds(base, RPC)], cols_v.at[par],
                            in_sems[par]).wait()
      pltpu.make_async_copy(w_hbm.at[pl.ds(base, RPC)], w_v.at[par],
                            in_sems[par]).wait()

    issue_chunk(0, 0)

    def do_chunk(ck, par):
      wait_chunk(ck, par)

      @pl.when(ck + 1 < total_chunks)
      def _():
        issue_chunk(ck + 1, 1 - par)
      compute_chunk(par)

    def compute_chunk(par):
      @plsc.parallel_loop(0, RPC * VPR, unroll=16,
                          carry=jnp.zeros((LANES,), jnp.int32))
      def ucntv(i, ucv):
        j = jnp.right_shift(i, 3)
        sl = pl.ds(jnp.bitwise_and(i, VPR - 1) * LANES, LANES)
        cvec = cols_v[par, j, sl]
        wvec = w_v[par, j, sl]
        iy = cvec - I
        mu = iy < 0
        gy = plsc.load_gather(y_tab, [jnp.where(mu, H, iy)])
        py[j, sl] = wvec * gy
        mui = mu.astype(jnp.int32)
        pos = ucv + plsc.cumsum(mui) - mui
        plsc.store_scatter(u_pos, [pos], i * LANES + iota, mask=mu)
        return ucv + plsc.all_reduce_population_count(mu)
      ucnt = jnp.max(ucntv)

      for j in range(RPC):
        pltpu.async_copy(py.at[j], acc_y.at[rows_v.at[par, j]], sc_sem,
                         add=True)

      RING = 16
      parv = jnp.full((LANES,), par, jnp.int32)

      def ublock(ub, carry):
        ubp = jnp.bitwise_and(ub, RING - 1)

        @pl.when(ub >= RING)
        def _wait_slot():
          pltpu.make_async_copy(pu_val.at[ubp],
                                acc_bu.at[pu_idx.at[ubp]], sc_sem).wait()
        for v in range(VPR):
          gl = v * LANES + iota
          eidx = ub * LANES + jnp.right_shift(gl, 3)
          valid = eidx < ucnt
          eidxc = jnp.where(valid, eidx, 0)
          pos2 = plsc.load_gather(u_pos, [eidxc])
          pj = jnp.right_shift(pos2, 7)
          pk = jnp.bitwise_and(pos2, 127)
          c2 = plsc.load_gather(cols_v, [parv, pj, pk])
          r2 = plsc.load_gather(rows_v, [parv, pj, pk])
          w2 = plsc.load_gather(w_v, [parv, pj, pk])
          g = plsc.load_gather(in_tab, [c2 * B + bl])
          pu_val[ubp, pl.ds(v * LANES, LANES)] = jnp.where(valid, w2 * g, 0.0)
          pu_idx[ubp, pl.ds(v * LANES, LANES)] = jnp.where(
              valid, bl * H + r2, DUMMY + gl)
        pltpu.async_copy(pu_val.at[ubp], acc_bu.at[pu_idx.at[ubp]], sc_sem,
                         add=True)
        return carry
      nub = (ucnt + (LANES - 1)) // LANES
      lax.fori_loop(0, nub, ublock, 0)

      def udrain(ub, _):
        ubp = jnp.bitwise_and(ub, RING - 1)
        pltpu.make_async_copy(pu_val.at[ubp],
                              acc_bu.at[pu_idx.at[ubp]], sc_sem).wait()
        return _
      lax.fori_loop(jnp.maximum(nub - RING, 0), nub, udrain, 0)

      for j in range(RPC):
        pltpu.make_async_copy(py.at[j], acc_y.at[rows_v.at[par, j]],
                              sc_sem).wait()

    def loop2(i, _):
      do_chunk(2 * i, 0)
      do_chunk(2 * i + 1, 1)
      return _
    lax.fori_loop(0, total_chunks // 2, loop2, 0)

    if tch:
      @pl.when(wid < tch)
      def _tail():
        tb = wid * RPC
        pltpu.sync_copy(rows_th.at[pl.ds(tb, RPC)], rows_v.at[0])
        pltpu.sync_copy(cols_th.at[pl.ds(tb, RPC)], cols_v.at[0])
        pltpu.sync_copy(w_th.at[pl.ds(tb, RPC)], w_v.at[0])
        compute_chunk(0)

    plsc.subcore_barrier()

    pltpu.sync_copy(acc_y.at[pl.ds(sub * ystride, ystride)],
                    accy_out.at[core, pl.ds(sub * ystride, ystride)])
    bout = (B * H) // NS
    half = H // 2
    pltpu.sync_copy(
        acc_bu.at[pl.ds(sub * bout, bout)],
        accbu_out.at[core * B + sub // 2,
                     pl.ds((sub % 2) * half, half)])

  return k(rows2d, cols2d, w2d, rows_t, cols_t, w_t, y_state, in_flat)


def _tc_finale(accy, accbu2, x2, tau2, bias2, act2, H, B, lo, osz):
  BH = 8192
  grid = (H // BH,)
  lo_blk = lo // BH
  lo_off = lo - lo_blk * BH

  def body(accy_ref, accbu_ref, x_ref, tau_ref, bias_ref, act_ref,
           nx_ref, out_ref, e_ref):
    accv = accy_ref[0:1, :] + accy_ref[1:2, :]
    act_in = accbu_ref[0:B] + accbu_ref[B:2 * B] + accv
    x = x_ref[...]
    dx = (act_in + bias_ref[...] - x) / tau_ref[...] * DT
    nx = x + dx
    nx_ref[...] = nx
    at = act_ref[...]
    ny = jnp.where(at == 0, jnp.tanh(nx),
                   jnp.where(at == 1, jax.nn.sigmoid(nx),
                             jnp.maximum(nx, 0.0)))

    @pl.when(pl.program_id(0) == lo_blk)
    def _():
      out_ref[...] = ny[:, lo_off:lo_off + osz]

    @pl.when(pl.program_id(0) == 0)
    def _():
      e_ref[...] = jnp.zeros((1, 1), jnp.float32)
    e_ref[...] += jnp.reshape(jnp.sum(jnp.abs(ny)), (1, 1))

  return pl.pallas_call(
      body,
      grid=grid,
      in_specs=[
          pl.BlockSpec((NC, BH), lambda i: (0, i)),
          pl.BlockSpec((NC * B, BH), lambda i: (0, i)),
          pl.BlockSpec((1, BH), lambda i: (0, i)),
          pl.BlockSpec((1, BH), lambda i: (0, i)),
          pl.BlockSpec((1, BH), lambda i: (0, i)),
          pl.BlockSpec((1, BH), lambda i: (0, i)),
      ],
      out_specs=[
          pl.BlockSpec((B, BH), lambda i: (0, i)),
          pl.BlockSpec((B, osz), lambda i: (0, 0)),
          pl.BlockSpec((1, 1), lambda i: (0, 0)),
      ],
      out_shape=[
          jax.ShapeDtypeStruct((B, H), jnp.float32),
          jax.ShapeDtypeStruct((B, osz), jnp.float32),
          jax.ShapeDtypeStruct((1, 1), jnp.float32),
      ],
  )(accy, accbu2, x2, tau2, bias2, act2)


def kernel(inputs, tau, bias, weight_values, x_state, y_state, rows, cols,
           act_types):
  B, I = inputs.shape
  H = tau.shape[0]
  nnz = rows.shape[0]
  NW = NC * NS

  chunks_pw = nnz // (NW * C)
  assert chunks_pw % 2 == 0
  main_n = NW * chunks_pw * C
  rem = nnz - main_n
  tch = -(-rem // C)
  pad = tch * C - rem
  rows32 = rows.astype(jnp.int32)
  cols32 = cols.astype(jnp.int32)
  w32 = weight_values.astype(jnp.float32)
  rows_m = rows32[:main_n].reshape(main_n // 128, 128)
  cols_m = cols32[:main_n].reshape(main_n // 128, 128)
  w_m = w32[:main_n].reshape(main_n // 128, 128)
  pad_rows = (jnp.arange(pad, dtype=jnp.int32) * 1031) % H
  rows_t = jnp.concatenate([rows32[main_n:], pad_rows]).reshape(-1, 128)
  cols_t = jnp.concatenate(
      [cols32[main_n:], jnp.full((pad,), I, jnp.int32)]).reshape(-1, 128)
  w_t = jnp.concatenate(
      [w32[main_n:], jnp.zeros((pad,), jnp.float32)]).reshape(-1, 128)
  in_flat = inputs.T.reshape(-1)

  accy, accbu = _sc_spmm(rows_m, cols_m, w_m, rows_t, cols_t, w_t, tch,
                         y_state, in_flat, H, B, I, chunks_pw)

  one = lambda v: v.reshape(1, H)
  lo = H - OUT_SZ - PAR_SZ
  new_x, outputs, e = _tc_finale(accy, accbu, one(x_state), one(tau),
                                 one(bias), one(act_types.astype(jnp.int32)),
                                 H, B, lo, OUT_SZ)
  energy = e[0, 0] * 0.001
  return (outputs, new_x, energy)
